# baseline (speedup 1.0000x reference)
import functools

import jax
import jax.numpy as jnp
from jax import lax
from jax.experimental import pallas as pl
from jax.experimental.pallas import tpu as pltpu
from jax.experimental.pallas import tpu_sc as plsc

_NUM_WORKERS = 32
_CHUNK = 64
_NBUF = 4
_VPAD = 1024


@functools.partial(jax.jit, static_argnums=(3,))
def _sc_embed_concat(x2, labels2, table, n_tokens):
    d_x = x2.shape[1]
    d_e = table.shape[1]
    d_out = d_x + d_e
    per_w = n_tokens // _NUM_WORKERS
    steps = per_w // _CHUNK
    rows_per_sub = _VPAD // 16
    mesh = plsc.VectorSubcoreMesh(core_axis_name="c", subcore_axis_name="s")

    @functools.partial(
        pl.kernel,
        mesh=mesh,
        out_type=jax.ShapeDtypeStruct((n_tokens, d_out), jnp.float32),
        scratch_types=[
            pltpu.VMEM((steps, _CHUNK), jnp.int32),
            pltpu.VMEM_SHARED((_VPAD, d_e), jnp.float32),
        ]
        + [pltpu.VMEM((_CHUNK, d_x), jnp.float32)] * _NBUF
        + [pltpu.VMEM((_CHUNK, d_e), jnp.float32)] * _NBUF
        + [pltpu.SemaphoreType.DMA] * (4 * _NBUF),
    )
    def k(x_hbm, lab_hbm, tab_hbm, out_hbm, idx_v, tab_sp, *bufs_and_sems):
        xbuf = bufs_and_sems[0:_NBUF]
        ebuf = bufs_and_sems[_NBUF:2 * _NBUF]
        sems = bufs_and_sems[2 * _NBUF:]
        sem_xin = sems[0:_NBUF]
        sem_gat = sems[_NBUF:2 * _NBUF]
        sem_xout = sems[2 * _NBUF:3 * _NBUF]
        sem_eout = sems[3 * _NBUF:]
        sid = lax.axis_index("s")
        wid = sid * 2 + lax.axis_index("c")
        base = wid * per_w

        def rows(j):
            return pl.ds(base + j * _CHUNK, _CHUNK)

        xin0 = pltpu.async_copy(x_hbm.at[rows(0), :], xbuf[0], sem_xin[0])

        tr = pl.ds(sid * rows_per_sub, rows_per_sub)
        pltpu.sync_copy(tab_hbm.at[tr, :], tab_sp.at[tr, :])
        pltpu.sync_copy(lab_hbm.at[pl.ds(wid * steps, steps), :], idx_v)
        plsc.subcore_barrier()

        def start_gat(j):
            p = j % _NBUF
            return pltpu.async_copy(tab_sp.at[idx_v.at[j]], ebuf[p], sem_gat[p])

        def start_reads(j):
            p = j % _NBUF
            xin = pltpu.async_copy(x_hbm.at[rows(j), :], xbuf[p], sem_xin[p])
            return xin, start_gat(j)

        reads = [None] * _NBUF
        writes = [None] * _NBUF
        reads[0] = (xin0, start_gat(0))
        for j in range(1, min(_NBUF - 1, steps)):
            reads[j % _NBUF] = start_reads(j)
        for j in range(steps):
            p = j % _NBUF
            nxt = j + _NBUF - 1
            if nxt < steps:
                q = nxt % _NBUF
                if writes[q] is not None:
                    writes[q][0].wait()
                    writes[q][1].wait()
                reads[q] = start_reads(nxt)
            xin, gat = reads[p]
            xin.wait()
            w_x = pltpu.async_copy(xbuf[p], out_hbm.at[rows(j), pl.ds(0, d_x)], sem_xout[p])
            gat.wait()
            w_e = pltpu.async_copy(ebuf[p], out_hbm.at[rows(j), pl.ds(d_x, d_e)], sem_eout[p])
            writes[p] = (w_x, w_e)
        for w in writes:
            if w is not None:
                w[0].wait()
                w[1].wait()

    return k(x2, labels2, table)


def kernel(x, sem_labels, embedding_weight, bbox):
    b, t, d_x = x.shape
    n = b * t
    x2 = x.reshape(n, d_x)
    labels2 = sem_labels.reshape(n // _CHUNK, _CHUNK).astype(jnp.int32)
    v, d_e = embedding_weight.shape
    table = jnp.pad(embedding_weight, ((0, _VPAD - v), (0, 0)))
    out2 = _sc_embed_concat(x2, labels2, table, n)
    return out2.reshape(b, t, d_x + d_e)


# speedup vs baseline: 1.0391x; 1.0391x over previous
import functools

import jax
import jax.numpy as jnp
from jax import lax
from jax.experimental import pallas as pl
from jax.experimental.pallas import tpu as pltpu
from jax.experimental.pallas import tpu_sc as plsc

_NUM_WORKERS = 32
_CHUNK = 128
_NBUF = 2
_VPAD = 1024


@functools.partial(jax.jit, static_argnums=(3,))
def _sc_embed_concat(x2, labels2, table, n_tokens):
    d_x = x2.shape[1]
    d_e = table.shape[1]
    d_out = d_x + d_e
    per_w = n_tokens // _NUM_WORKERS
    steps = per_w // _CHUNK
    rows_per_sub = _VPAD // 16
    mesh = plsc.VectorSubcoreMesh(core_axis_name="c", subcore_axis_name="s")

    @functools.partial(
        pl.kernel,
        mesh=mesh,
        out_type=jax.ShapeDtypeStruct((n_tokens, d_out), jnp.float32),
        scratch_types=[
            pltpu.VMEM((steps, _CHUNK), jnp.int32),
            pltpu.VMEM_SHARED((table.shape[0], d_e), jnp.float32),
        ]
        + [pltpu.VMEM((_CHUNK, d_x), jnp.float32)] * _NBUF
        + [pltpu.VMEM((_CHUNK, d_e), jnp.float32)] * _NBUF
        + [pltpu.SemaphoreType.DMA] * (4 * _NBUF),
    )
    def k(x_hbm, lab_hbm, tab_hbm, out_hbm, idx_v, tab_sp, *bufs_and_sems):
        xbuf = bufs_and_sems[0:_NBUF]
        ebuf = bufs_and_sems[_NBUF:2 * _NBUF]
        sems = bufs_and_sems[2 * _NBUF:]
        sem_xin = sems[0:_NBUF]
        sem_gat = sems[_NBUF:2 * _NBUF]
        sem_xout = sems[2 * _NBUF:3 * _NBUF]
        sem_eout = sems[3 * _NBUF:]
        sid = lax.axis_index("s")
        wid = sid * 2 + lax.axis_index("c")
        base = wid * per_w

        def rows(j):
            return pl.ds(base + j * _CHUNK, _CHUNK)

        xin0 = pltpu.async_copy(x_hbm.at[rows(0), :], xbuf[0], sem_xin[0])

        n_tab = tab_hbm.shape[0]
        last_rows = n_tab - 15 * rows_per_sub

        @pl.when(sid < 15)
        def _():
            tr = pl.ds(sid * rows_per_sub, rows_per_sub)
            pltpu.sync_copy(tab_hbm.at[tr, :], tab_sp.at[tr, :])

        @pl.when(sid == 15)
        def _():
            tr = pl.ds(15 * rows_per_sub, last_rows)
            pltpu.sync_copy(tab_hbm.at[tr, :], tab_sp.at[tr, :])
        pltpu.sync_copy(lab_hbm.at[pl.ds(wid * steps, steps), :], idx_v)
        plsc.subcore_barrier()

        def start_gat(j):
            p = j % _NBUF
            return pltpu.async_copy(tab_sp.at[idx_v.at[j]], ebuf[p], sem_gat[p])

        def start_reads(j):
            p = j % _NBUF
            xin = pltpu.async_copy(x_hbm.at[rows(j), :], xbuf[p], sem_xin[p])
            return xin, start_gat(j)

        reads = [None] * _NBUF
        writes = [None] * _NBUF
        reads[0] = (xin0, start_gat(0))
        for j in range(1, min(_NBUF - 1, steps)):
            reads[j % _NBUF] = start_reads(j)
        for j in range(steps):
            p = j % _NBUF
            nxt = j + _NBUF - 1
            if nxt < steps:
                q = nxt % _NBUF
                if writes[q] is not None:
                    writes[q][0].wait()
                    writes[q][1].wait()
                reads[q] = start_reads(nxt)
            xin, gat = reads[p]
            xin.wait()
            w_x = pltpu.async_copy(xbuf[p], out_hbm.at[rows(j), pl.ds(0, d_x)], sem_xout[p])
            gat.wait()
            w_e = pltpu.async_copy(ebuf[p], out_hbm.at[rows(j), pl.ds(d_x, d_e)], sem_eout[p])
            writes[p] = (w_x, w_e)
        for w in writes:
            if w is not None:
                w[0].wait()
                w[1].wait()

    return k(x2, labels2, table)


def kernel(x, sem_labels, embedding_weight, bbox):
    b, t, d_x = x.shape
    n = b * t
    x2 = x.reshape(n, d_x)
    labels2 = sem_labels.reshape(n // _CHUNK, _CHUNK).astype(jnp.int32)
    d_e = embedding_weight.shape[1]
    out2 = _sc_embed_concat(x2, labels2, embedding_weight, n)
    return out2.reshape(b, t, d_x + d_e)


# speedup vs baseline: 1.0415x; 1.0022x over previous
import functools

import jax
import jax.numpy as jnp
from jax import lax
from jax.experimental import pallas as pl
from jax.experimental.pallas import tpu as pltpu
from jax.experimental.pallas import tpu_sc as plsc

_NUM_WORKERS = 32
_CHUNK = 128
_NBUF = 2
_VPAD = 1024


@functools.partial(jax.jit, static_argnums=(3,))
def _sc_embed_concat(x2, labels2, table, n_tokens):
    d_x = x2.shape[1]
    d_e = table.shape[1]
    d_out = d_x + d_e
    per_w = n_tokens // _NUM_WORKERS
    steps = per_w // _CHUNK
    rows_per_sub = _VPAD // 16
    mesh = plsc.VectorSubcoreMesh(core_axis_name="c", subcore_axis_name="s")

    @functools.partial(
        pl.kernel,
        mesh=mesh,
        out_type=jax.ShapeDtypeStruct((n_tokens, d_out), jnp.float32),
        scratch_types=[
            pltpu.VMEM((steps, _CHUNK), jnp.int32),
            pltpu.VMEM_SHARED((table.shape[0], d_e), jnp.float32),
        ]
        + [pltpu.VMEM((_CHUNK, d_x), jnp.float32)] * _NBUF
        + [pltpu.VMEM((_CHUNK, d_e), jnp.float32)] * _NBUF
        + [pltpu.SemaphoreType.DMA] * (4 * _NBUF),
    )
    def k(x_hbm, lab_hbm, tab_hbm, out_hbm, idx_v, tab_sp, *bufs_and_sems):
        xbuf = bufs_and_sems[0:_NBUF]
        ebuf = bufs_and_sems[_NBUF:2 * _NBUF]
        sems = bufs_and_sems[2 * _NBUF:]
        sem_xin = sems[0:_NBUF]
        sem_gat = sems[_NBUF:2 * _NBUF]
        sem_xout = sems[2 * _NBUF:3 * _NBUF]
        sem_eout = sems[3 * _NBUF:]
        sid = lax.axis_index("s")
        wid = sid * 2 + lax.axis_index("c")
        base = wid * per_w

        def rows(j):
            return pl.ds(base + j * _CHUNK, _CHUNK)

        xin0 = pltpu.async_copy(x_hbm.at[rows(0), :], xbuf[0], sem_xin[0])

        n_tab = tab_hbm.shape[0]
        last_rows = n_tab - 15 * rows_per_sub

        @pl.when(sid < 15)
        def _():
            tr = pl.ds(sid * rows_per_sub, rows_per_sub)
            pltpu.sync_copy(tab_hbm.at[tr, :], tab_sp.at[tr, :])

        @pl.when(sid == 15)
        def _():
            tr = pl.ds(15 * rows_per_sub, last_rows)
            pltpu.sync_copy(tab_hbm.at[tr, :], tab_sp.at[tr, :])
        pltpu.sync_copy(lab_hbm.at[pl.ds(wid * steps, steps), :], idx_v)
        plsc.subcore_barrier()

        def start_gat(j):
            p = j % _NBUF
            return pltpu.async_copy(tab_sp.at[idx_v.at[j]], ebuf[p], sem_gat[p])

        def start_reads(j):
            p = j % _NBUF
            xin = pltpu.async_copy(x_hbm.at[rows(j), :], xbuf[p], sem_xin[p])
            return xin, start_gat(j)

        reads = [None] * _NBUF
        writes = [None] * _NBUF
        reads[0] = (xin0, start_gat(0))
        for j in range(1, min(_NBUF - 1, steps)):
            reads[j % _NBUF] = start_reads(j)
        for j in range(steps):
            p = j % _NBUF
            nxt = j + _NBUF - 1
            if nxt < steps:
                q = nxt % _NBUF
                if writes[q] is not None:
                    writes[q][0].wait()
                    writes[q][1].wait()
                reads[q] = start_reads(nxt)
            xin, gat = reads[p]
            gat.wait()
            w_e = pltpu.async_copy(ebuf[p], out_hbm.at[rows(j), pl.ds(d_x, d_e)], sem_eout[p])
            xin.wait()
            w_x = pltpu.async_copy(xbuf[p], out_hbm.at[rows(j), pl.ds(0, d_x)], sem_xout[p])
            writes[p] = (w_x, w_e)
        for w in writes:
            if w is not None:
                w[0].wait()
                w[1].wait()

    return k(x2, labels2, table)


def kernel(x, sem_labels, embedding_weight, bbox):
    b, t, d_x = x.shape
    n = b * t
    x2 = x.reshape(n, d_x)
    labels2 = sem_labels.reshape(n // _CHUNK, _CHUNK).astype(jnp.int32)
    d_e = embedding_weight.shape[1]
    out2 = _sc_embed_concat(x2, labels2, embedding_weight, n)
    return out2.reshape(b, t, d_x + d_e)


# speedup vs baseline: 1.0420x; 1.0005x over previous
import functools

import jax
import jax.numpy as jnp
from jax import lax
from jax.experimental import pallas as pl
from jax.experimental.pallas import tpu as pltpu
from jax.experimental.pallas import tpu_sc as plsc

_NUM_WORKERS = 32
_CHUNK = 128
_NBUF = 2
_STAGE_SPLIT = 64


@functools.partial(jax.jit, static_argnums=(3,))
def _sc_embed_concat(x2, labels2, table, n_tokens):
    d_x = x2.shape[1]
    d_e = table.shape[1]
    d_out = d_x + d_e
    per_w = n_tokens // _NUM_WORKERS
    steps = per_w // _CHUNK
    rows_per_sub = _STAGE_SPLIT
    mesh = plsc.VectorSubcoreMesh(core_axis_name="c", subcore_axis_name="s")

    @functools.partial(
        pl.kernel,
        mesh=mesh,
        out_type=jax.ShapeDtypeStruct((n_tokens, d_out), jnp.float32),
        scratch_types=[
            pltpu.VMEM((steps, _CHUNK), jnp.int32),
            pltpu.VMEM_SHARED((table.shape[0], d_e), jnp.float32),
        ]
        + [pltpu.VMEM((_CHUNK, d_x), jnp.float32)] * _NBUF
        + [pltpu.VMEM((_CHUNK, d_e), jnp.float32)] * _NBUF
        + [pltpu.SemaphoreType.DMA] * (4 * _NBUF),
    )
    def k(x_hbm, lab_hbm, tab_hbm, out_hbm, idx_v, tab_sp, *bufs_and_sems):
        xbuf = bufs_and_sems[0:_NBUF]
        ebuf = bufs_and_sems[_NBUF:2 * _NBUF]
        sems = bufs_and_sems[2 * _NBUF:]
        sem_xin = sems[0:_NBUF]
        sem_gat = sems[_NBUF:2 * _NBUF]
        sem_xout = sems[2 * _NBUF:3 * _NBUF]
        sem_eout = sems[3 * _NBUF:]
        sid = lax.axis_index("s")
        wid = sid * 2 + lax.axis_index("c")
        base = wid * per_w

        def rows(j):
            return pl.ds(base + j * _CHUNK, _CHUNK)

        xin0 = pltpu.async_copy(x_hbm.at[rows(0), :], xbuf[0], sem_xin[0])

        n_tab = tab_hbm.shape[0]
        last_rows = n_tab - 15 * rows_per_sub

        @pl.when(sid < 15)
        def _():
            tr = pl.ds(sid * rows_per_sub, rows_per_sub)
            pltpu.sync_copy(tab_hbm.at[tr, :], tab_sp.at[tr, :])

        @pl.when(sid == 15)
        def _():
            tr = pl.ds(15 * rows_per_sub, last_rows)
            pltpu.sync_copy(tab_hbm.at[tr, :], tab_sp.at[tr, :])
        pltpu.sync_copy(lab_hbm.at[pl.ds(wid * steps, steps), :], idx_v)
        plsc.subcore_barrier()

        def start_gat(j):
            p = j % _NBUF
            return pltpu.async_copy(tab_sp.at[idx_v.at[j]], ebuf[p], sem_gat[p])

        def start_reads(j):
            p = j % _NBUF
            xin = pltpu.async_copy(x_hbm.at[rows(j), :], xbuf[p], sem_xin[p])
            return xin, start_gat(j)

        reads = [None] * _NBUF
        writes = [None] * _NBUF
        reads[0] = (xin0, start_gat(0))
        for j in range(1, min(_NBUF - 1, steps)):
            reads[j % _NBUF] = start_reads(j)
        for j in range(steps):
            p = j % _NBUF
            nxt = j + _NBUF - 1
            if nxt < steps:
                q = nxt % _NBUF
                if writes[q] is not None:
                    writes[q][0].wait()
                    writes[q][1].wait()
                reads[q] = start_reads(nxt)
            xin, gat = reads[p]
            gat.wait()
            w_e = pltpu.async_copy(ebuf[p], out_hbm.at[rows(j), pl.ds(d_x, d_e)], sem_eout[p])
            xin.wait()
            w_x = pltpu.async_copy(xbuf[p], out_hbm.at[rows(j), pl.ds(0, d_x)], sem_xout[p])
            writes[p] = (w_x, w_e)
        for w in writes:
            if w is not None:
                w[0].wait()
                w[1].wait()

    return k(x2, labels2, table)


def kernel(x, sem_labels, embedding_weight, bbox):
    b, t, d_x = x.shape
    n = b * t
    x2 = x.reshape(n, d_x)
    labels2 = sem_labels.reshape(n // _CHUNK, _CHUNK).astype(jnp.int32)
    d_e = embedding_weight.shape[1]
    out2 = _sc_embed_concat(x2, labels2, embedding_weight, n)
    return out2.reshape(b, t, d_x + d_e)
